# baseline (speedup 1.0000x reference)
import functools
import jax
import jax.numpy as jnp
from jax import lax
from jax.experimental import pallas as pl
from jax.experimental.pallas import tpu as pltpu
from jax.experimental.pallas import tpu_sc as plsc

_B, _S, _D = 16, 2048, 1024
_CH = 16
_NCH = 1024 // _CH


def _acc_half(acc_ref, buf, mv16, mof, off):
    n = 32

    def tbody(t, accs):
        m_t = mv16[pl.ds(16 * (mof + t), 16)]
        return tuple(
            accs[i] + buf[t, pl.ds(16 * (off + i), 16)] * m_t
            for i in range(n))

    accs0 = tuple(acc_ref[pl.ds(16 * (off + i), 16)] for i in range(n))
    accs = lax.fori_loop(0, _CH, tbody, accs0)
    for i in range(n):
        acc_ref[pl.ds(16 * (off + i), 16)] = accs[i]


def _sc_body(x_hbm, mf_hbm, o_hbm, bufs, acc, partner, mean, rep,
             mv16, psum, insem, outsems, drainsem):
    c = lax.axis_index("c")
    s = lax.axis_index("s")
    lb = s // 2
    b = c * 8 + lb
    half = s % 2
    tok0 = half * 1024

    pltpu.sync_copy(mf_hbm.at[b], mv16)

    for i in range(_D // 16):
        acc[pl.ds(16 * i, 16)] = jnp.zeros((16,), jnp.float32)

    def chunk(cidx, p):
        t0 = tok0 + cidx * _CH

        @pl.when(cidx >= 2)
        def _():
            pltpu.make_async_copy(
                bufs.at[p],
                o_hbm.at[b, pl.ds(tok0 + (cidx - 2) * _CH, _CH), pl.ds(0, _D)],
                outsems.at[p]).wait()

        pltpu.sync_copy(x_hbm.at[b, pl.ds(t0, _CH)], bufs.at[p])
        _acc_half(acc, bufs.at[p], mv16, tok0 + cidx * _CH, 0)
        _acc_half(acc, bufs.at[p], mv16, tok0 + cidx * _CH, 32)
        pltpu.make_async_copy(
            bufs.at[p],
            o_hbm.at[b, pl.ds(t0, _CH), pl.ds(0, _D)],
            outsems.at[p]).start()

    def gbody(g, _):
        chunk(2 * g, 0)
        chunk(2 * g + 1, 1)
        return 0

    lax.fori_loop(0, _NCH // 2, gbody, 0)
    for p in range(2):
        pltpu.make_async_copy(
            bufs.at[p],
            o_hbm.at[b, pl.ds(tok0 + (_NCH - 2 + p) * _CH, _CH), pl.ds(0, _D)],
            outsems.at[p]).wait()

    def cbody(t, v):
        return v + mv16[pl.ds(16 * t, 16)]
    tot_cnt = lax.fori_loop(0, 2048, cbody, jnp.zeros((16,), jnp.float32))

    pltpu.sync_copy(acc, psum.at[lb, half])
    plsc.subcore_barrier()
    pltpu.sync_copy(psum.at[lb, 1 - half], partner)
    for i in range(_D // 16):
        mean[pl.ds(16 * i, 16)] = (
            (acc[pl.ds(16 * i, 16)] + partner[pl.ds(16 * i, 16)]) / tot_cnt)

    def rbody(r, _):
        for i in range(_D // 16):
            rep[r, pl.ds(16 * i, 16)] = mean[pl.ds(16 * i, 16)]
        return 0

    lax.fori_loop(0, 16, rbody, 0)

    for w in range(1024 // 16):
        pltpu.make_async_copy(
            rep,
            o_hbm.at[b, pl.ds(tok0 + w * 16, 16), pl.ds(_D, _D)],
            drainsem).start()
    for w in range(1024 // 16):
        pltpu.make_async_copy(
            rep,
            o_hbm.at[b, pl.ds(tok0 + w * 16, 16), pl.ds(_D, _D)],
            drainsem).wait()


def kernel(inputs, mask):
    B, S, D = inputs.shape
    mf = jnp.broadcast_to(
        mask.astype(inputs.dtype)[:, :, None], (B, S, 16)).reshape(B, S * 16)

    mesh = plsc.VectorSubcoreMesh(core_axis_name="c", subcore_axis_name="s")
    run = functools.partial(
        pl.kernel,
        mesh=mesh,
        out_type=jax.ShapeDtypeStruct((B, S, 2 * D), inputs.dtype),
        scratch_types=[
            pltpu.VMEM((2, _CH, _D), jnp.float32),
            pltpu.VMEM((_D,), jnp.float32),
            pltpu.VMEM((_D,), jnp.float32),
            pltpu.VMEM((_D,), jnp.float32),
            pltpu.VMEM((16, _D), jnp.float32),
            pltpu.VMEM((32768,), jnp.float32),
            pltpu.VMEM_SHARED((8, 2, _D), jnp.float32),
            pltpu.SemaphoreType.DMA,
            pltpu.SemaphoreType.DMA((2,)),
            pltpu.SemaphoreType.DMA,
        ],
    )(_sc_body)
    return run(inputs, mf)


# speedup vs baseline: 1.8529x; 1.8529x over previous
import jax
import jax.numpy as jnp
from jax.experimental import pallas as pl
from jax.experimental.pallas import tpu as pltpu


def _in_copy(x_hbm, ob, insems, b, buf, D):
    return pltpu.make_async_copy(
        x_hbm.at[b], ob.at[buf, :, pl.ds(0, D)], insems.at[buf])


def _out_copy(o_hbm, ob, outsems, b, buf):
    return pltpu.make_async_copy(ob.at[buf], o_hbm.at[b], outsems.at[buf])


def _body(x_hbm, mf_hbm, o_hbm, ob, mv, insems, outsems, msem):
    B, S, D = x_hbm.shape

    mcp = pltpu.make_async_copy(mf_hbm, mv, msem)
    mcp.start()
    _in_copy(x_hbm, ob, insems, 0, 0, D).start()
    _in_copy(x_hbm, ob, insems, 1, 1, D).start()
    mcp.wait()

    def step(b, _):
        buf = jax.lax.rem(b, 3)
        _in_copy(x_hbm, ob, insems, b, buf, D).wait()

        @pl.when(b >= 1)
        def _():
            _out_copy(o_hbm, ob, outsems, b - 1, jax.lax.rem(b + 2, 3)).wait()

        @pl.when(b + 2 < B)
        def _():
            _in_copy(x_hbm, ob, insems, b + 2, jax.lax.rem(b + 2, 3), D).start()

        x = ob[buf, :, pl.ds(0, D)]
        m1 = mv[b]
        s = jax.lax.dot_general(
            m1, x, (((1,), (0,)), ((), ())),
            preferred_element_type=jnp.float32,
            precision=jax.lax.Precision.DEFAULT)
        cnt = jnp.sum(m1)
        mean = s / cnt
        ob[buf, :, pl.ds(D, D)] = jnp.broadcast_to(mean, (S, D))

        _out_copy(o_hbm, ob, outsems, b, buf).start()
        return 0

    jax.lax.fori_loop(0, B, step, 0)
    _out_copy(o_hbm, ob, outsems, B - 1, jax.lax.rem(B - 1, 3)).wait()


def kernel(inputs, mask):
    B, S, D = inputs.shape
    mf = mask.astype(inputs.dtype).reshape(B, 1, S)

    out = pl.pallas_call(
        _body,
        in_specs=[
            pl.BlockSpec(memory_space=pltpu.HBM),
            pl.BlockSpec(memory_space=pltpu.HBM),
        ],
        out_specs=pl.BlockSpec(memory_space=pltpu.HBM),
        out_shape=jax.ShapeDtypeStruct((B, S, 2 * D), inputs.dtype),
        scratch_shapes=[
            pltpu.VMEM((3, S, 2 * D), inputs.dtype),
            pltpu.VMEM((B, 1, S), inputs.dtype),
            pltpu.SemaphoreType.DMA((3,)),
            pltpu.SemaphoreType.DMA((3,)),
            pltpu.SemaphoreType.DMA,
        ],
        compiler_params=pltpu.CompilerParams(
            vmem_limit_bytes=60 * 1024 * 1024,
        ),
    )(inputs, mf)
    return out


# speedup vs baseline: 1.9993x; 1.0790x over previous
import jax
import jax.numpy as jnp
from jax.experimental import pallas as pl
from jax.experimental.pallas import tpu as pltpu


def _in_copy(x_hbm, ob, insems, b, buf, D):
    return pltpu.make_async_copy(
        x_hbm.at[b], ob.at[buf, :, pl.ds(0, D)], insems.at[buf])


def _out_copy(o_hbm, ob, outsems, b, buf):
    return pltpu.make_async_copy(ob.at[buf], o_hbm.at[b], outsems.at[buf])


def _body(x_hbm, mf_hbm, o_hbm, ob, mv, insems, outsems, msem):
    B, S, D = x_hbm.shape

    mcp = pltpu.make_async_copy(mf_hbm, mv, msem)
    mcp.start()
    _in_copy(x_hbm, ob, insems, 0, 0, D).start()
    _in_copy(x_hbm, ob, insems, 1, 1, D).start()
    mcp.wait()

    def step(b, _):
        buf = jax.lax.rem(b, 3)
        _in_copy(x_hbm, ob, insems, b, buf, D).wait()

        x = ob[buf, :, pl.ds(0, D)]
        m1 = mv[b]
        s = jax.lax.dot_general(
            m1, x, (((1,), (0,)), ((), ())),
            preferred_element_type=jnp.float32,
            precision=jax.lax.Precision.DEFAULT)
        cnt = jnp.sum(m1)
        mean = s / cnt
        ob[buf, :, pl.ds(D, D)] = jnp.broadcast_to(mean, (S, D))

        @pl.when(b >= 1)
        def _():
            _out_copy(o_hbm, ob, outsems, b - 1, jax.lax.rem(b + 2, 3)).wait()

        _out_copy(o_hbm, ob, outsems, b, buf).start()

        @pl.when(b + 2 < B)
        def _():
            _in_copy(x_hbm, ob, insems, b + 2, jax.lax.rem(b + 2, 3), D).start()
        return 0

    jax.lax.fori_loop(0, B, step, 0)
    _out_copy(o_hbm, ob, outsems, B - 1, jax.lax.rem(B - 1, 3)).wait()


def kernel(inputs, mask):
    B, S, D = inputs.shape
    mf = mask.astype(inputs.dtype).reshape(B, 1, S)

    out = pl.pallas_call(
        _body,
        in_specs=[
            pl.BlockSpec(memory_space=pltpu.HBM),
            pl.BlockSpec(memory_space=pltpu.HBM),
        ],
        out_specs=pl.BlockSpec(memory_space=pltpu.HBM),
        out_shape=jax.ShapeDtypeStruct((B, S, 2 * D), inputs.dtype),
        scratch_shapes=[
            pltpu.VMEM((3, S, 2 * D), inputs.dtype),
            pltpu.VMEM((B, 1, S), inputs.dtype),
            pltpu.SemaphoreType.DMA((3,)),
            pltpu.SemaphoreType.DMA((3,)),
            pltpu.SemaphoreType.DMA,
        ],
        compiler_params=pltpu.CompilerParams(
            vmem_limit_bytes=60 * 1024 * 1024,
        ),
    )(inputs, mf)
    return out
